# Initial kernel scaffold; baseline (speedup 1.0000x reference)
import jax
import jax.numpy as jnp
from jax.experimental import pallas as pl


def kernel(grid, row_emb, col_emb, tok_emb, ln_w, ln_b):
    raise NotImplementedError("write your pallas kernel here")


# speedup vs baseline: 1.8234x; 1.8234x over previous
import jax
import jax.numpy as jnp
from jax.experimental import pallas as pl

B, NF, H, W, D = 16, 4, 24, 24, 768
ROWS_PER_BLK = 8
NBLK = H // ROWS_PER_BLK
TOK_PER_BLK = ROWS_PER_BLK * W
EPS = 1e-12


def _body(g_ref, row_ref, col_ref, tok_ref, w_ref, b_ref, o_ref):
    c = pl.program_id(1)
    g = g_ref[0]
    m = (g[0] + g[1] + g[2] + g[3]) * 0.25
    r = row_ref[pl.ds(c * ROWS_PER_BLK, ROWS_PER_BLK), :]
    cc = col_ref[:W, :]
    t = tok_ref[0, :]
    x = m + r[:, None, :] + cc[None, :, :] + t[None, None, :]
    mu = jnp.mean(x, axis=-1, keepdims=True)
    xc = x - mu
    var = jnp.mean(xc * xc, axis=-1, keepdims=True)
    y = xc * jax.lax.rsqrt(var + EPS)
    y = y * w_ref[0][None, None, :] + b_ref[0][None, None, :]
    o_ref[0] = y.reshape(TOK_PER_BLK, D)


def kernel(grid, row_emb, col_emb, tok_emb, ln_w, ln_b):
    ln_w2 = ln_w.reshape(1, D)
    ln_b2 = ln_b.reshape(1, D)
    out = pl.pallas_call(
        _body,
        grid=(B, NBLK),
        in_specs=[
            pl.BlockSpec((1, NF, ROWS_PER_BLK, W, D),
                         lambda b, c: (b, 0, c, 0, 0)),
            pl.BlockSpec((32, D), lambda b, c: (0, 0)),
            pl.BlockSpec((32, D), lambda b, c: (0, 0)),
            pl.BlockSpec((1, D), lambda b, c: (0, 0)),
            pl.BlockSpec((1, D), lambda b, c: (0, 0)),
            pl.BlockSpec((1, D), lambda b, c: (0, 0)),
        ],
        out_specs=pl.BlockSpec((1, TOK_PER_BLK, D), lambda b, c: (b, c, 0)),
        out_shape=jax.ShapeDtypeStruct((B, H * W, D), jnp.float32),
    )(grid, row_emb, col_emb, tok_emb, ln_w2, ln_b2)
    return out
